# baseline (speedup 1.0000x reference)
import functools
import math

import jax
import jax.numpy as jnp
from jax import lax
from jax.experimental import pallas as pl
from jax.experimental.pallas import tpu as pltpu
from jax.experimental.pallas import tpu_sc as plsc

D_MODEL = 64
LANES = 16
NUM_CORES = 2
NUM_SUBCORES = 16
NUM_WORKERS = NUM_CORES * NUM_SUBCORES
SCALE = math.sqrt(D_MODEL)

CHUNK = 128


def _emb_body(nchunk, idx_hbm, table_hbm, out_hbm, idx_v, rows_v, sem):
    wid = lax.axis_index("s") * NUM_CORES + lax.axis_index("c")
    base = wid * nchunk * CHUNK

    pltpu.sync_copy(idx_hbm.at[wid], idx_v)

    def chunk_body(g, _):
        pltpu.async_copy(table_hbm.at[idx_v.at[g]], rows_v, sem).wait()

        def row_body(i, _):
            for j in range(D_MODEL // LANES):
                sl = pl.ds(j * LANES, LANES)
                rows_v[i, sl] = rows_v[i, sl] * SCALE
            return ()

        lax.fori_loop(0, CHUNK, row_body, (), unroll=4)

        pltpu.sync_copy(rows_v, out_hbm.at[pl.ds(base + g * CHUNK, CHUNK)])
        return ()

    lax.fori_loop(0, nchunk, chunk_body, ())


@functools.partial(jax.jit, static_argnames=("nchunk",))
def _emb_call(idx, table, nchunk):
    mesh = plsc.VectorSubcoreMesh(core_axis_name="c", subcore_axis_name="s")
    return pl.kernel(
        functools.partial(_emb_body, nchunk),
        mesh=mesh,
        out_type=jax.ShapeDtypeStruct((NUM_WORKERS * nchunk * CHUNK, D_MODEL),
                                      jnp.float32),
        scratch_types=[
            pltpu.VMEM((nchunk, CHUNK), jnp.int32),
            pltpu.VMEM((CHUNK, D_MODEL), jnp.float32),
            pltpu.SemaphoreType.DMA,
        ],
        compiler_params=pltpu.CompilerParams(use_tc_tiling_on_sc=False),
    )(idx, table)


def kernel(x, table):
    b, s = x.shape
    total = b * s
    nchunk = total // (NUM_WORKERS * CHUNK)
    idx = x.reshape(NUM_WORKERS, nchunk, CHUNK)
    out = _emb_call(idx, table, nchunk)
    return out.reshape(b, s, D_MODEL)


# speedup vs baseline: 1.1629x; 1.1629x over previous
import functools
import math

import jax
import jax.numpy as jnp
from jax import lax
from jax.experimental import pallas as pl
from jax.experimental.pallas import tpu as pltpu
from jax.experimental.pallas import tpu_sc as plsc

D_MODEL = 64
LANES = 16
NUM_CORES = 2
NUM_SUBCORES = 16
NUM_WORKERS = NUM_CORES * NUM_SUBCORES
SCALE = math.sqrt(D_MODEL)

CHUNK = 128
GPC = 2
GROUP = CHUNK * GPC
NBUF = 2


def _emb_body(ngroups, idx_hbm, table_hbm, out_hbm,
              idx_v, in0, in1, out0, out1, gs0, gs1, ss0, ss1):
    wid = lax.axis_index("s") * NUM_CORES + lax.axis_index("c")
    row_base = wid * ngroups * GROUP

    ins = (in0, in1)
    outs = (out0, out1)
    gsems = (gs0, gs1)
    ssems = (ss0, ss1)

    pltpu.sync_copy(idx_hbm.at[wid], idx_v)

    def gather_desc(g, b, h):
        return pltpu.make_async_copy(
            table_hbm.at[idx_v.at[GPC * g + h]],
            ins[b].at[pl.ds(h * CHUNK, CHUNK)],
            gsems[b])

    def store_desc(g, b):
        return pltpu.make_async_copy(
            outs[b],
            out_hbm.at[pl.ds((row_base + g * GROUP) * D_MODEL,
                             GROUP * D_MODEL)],
            ssems[b])

    def issue_gather(g, b):
        for h in range(GPC):
            gather_desc(g, b, h).start()

    for b in range(NBUF):
        issue_gather(b, b)

    def outer(g0, _):
        for b in range(NBUF):
            g = g0 * NBUF + b
            inb, outb = ins[b], outs[b]
            for h in range(GPC):
                gather_desc(g, b, h).wait()
            @pl.when(g >= NBUF)
            def _():
                store_desc(g - NBUF, b).wait()

            def scale_body(i, _):
                for u in range(4):
                    row = i * 4 + u
                    for j in range(D_MODEL // LANES):
                        outb[pl.ds(row * D_MODEL + j * LANES, LANES)] = (
                            inb[row, pl.ds(j * LANES, LANES)] * SCALE)
                return ()

            lax.fori_loop(0, GROUP // 4, scale_body, ())

            @pl.when(g + NBUF < ngroups)
            def _():
                issue_gather(g + NBUF, b)
            store_desc(g, b).start()
        return ()

    lax.fori_loop(0, ngroups // NBUF, outer, ())

    for b in range(NBUF):
        store_desc(ngroups - NBUF + b, b).wait()


@functools.partial(jax.jit, static_argnames=("ngroups",))
def _emb_call(idx, table, ngroups):
    mesh = plsc.VectorSubcoreMesh(core_axis_name="c", subcore_axis_name="s")
    n_out = NUM_WORKERS * ngroups * GROUP * D_MODEL
    return pl.kernel(
        functools.partial(_emb_body, ngroups),
        mesh=mesh,
        out_type=jax.ShapeDtypeStruct((n_out,), jnp.float32),
        scratch_types=[
            pltpu.VMEM((ngroups * GPC, CHUNK), jnp.int32),
            pltpu.VMEM((GROUP, D_MODEL), jnp.float32),
            pltpu.VMEM((GROUP, D_MODEL), jnp.float32),
            pltpu.VMEM((GROUP * D_MODEL,), jnp.float32),
            pltpu.VMEM((GROUP * D_MODEL,), jnp.float32),
            pltpu.SemaphoreType.DMA,
            pltpu.SemaphoreType.DMA,
            pltpu.SemaphoreType.DMA,
            pltpu.SemaphoreType.DMA,
        ],
        compiler_params=pltpu.CompilerParams(use_tc_tiling_on_sc=False),
    )(idx, table)


def kernel(x, table):
    b, s = x.shape
    total = b * s
    ngroups = total // (NUM_WORKERS * GROUP)
    idx = x.reshape(NUM_WORKERS, ngroups * GPC, CHUNK)
    out = _emb_call(idx, table, ngroups)
    return out.reshape(b, s, D_MODEL)
